# baseline (speedup 1.0000x reference)
import functools

import jax
import jax.numpy as jnp
import numpy as np
from jax.experimental import pallas as pl
from jax.experimental.pallas import tpu as pltpu

_TINY = np.float32(np.finfo(np.float32).tiny)
_NEG_INF = np.float32(-np.inf)
_INT_MAX = np.int32(np.iinfo(np.int32).max)


def _gumbel_from_counter(t):
    ks1 = jnp.uint32(42)
    ks2 = jnp.uint32(0 ^ 42 ^ 0x1BD11BDA)
    ks0 = jnp.uint32(0)
    inj = ((ks1, ks2 + jnp.uint32(1)), (ks2, ks0 + jnp.uint32(2)),
           (ks0, ks1 + jnp.uint32(3)), (ks1, ks2 + jnp.uint32(4)),
           (ks2, ks0 + jnp.uint32(5)))
    rots = ((13, 15, 26, 6), (17, 29, 16, 24))
    x0 = t
    x1 = ((t << jnp.uint32(13)) | (t >> jnp.uint32(19))) ^ t
    first = True
    for g in range(5):
        for r in rots[g & 1]:
            if first:
                first = False
                continue
            x0 = x0 + x1
            x1 = ((x1 << jnp.uint32(r)) | (x1 >> jnp.uint32(32 - r))) ^ x0
        a, b = inj[g]
        x0 = x0 + a
        x1 = x1 + b
    bits = x0 ^ x1
    fb = (bits >> jnp.uint32(9)) | jnp.uint32(0x3F800000)
    f = jax.lax.bitcast_convert_type(fb, jnp.float32) - jnp.float32(1.0)
    u = jnp.maximum(_TINY, f)
    return -jnp.log(-jnp.log(u))


def _body(x_hbm, base_ref, xt_ref, o_ref, xbuf, acc, blk, sem, *, vb, nf,
          tail):
    def start_copy(step, slot):
        pltpu.make_async_copy(x_hbm.at[:, pl.ds(step * vb, vb)],
                              xbuf.at[slot], sem.at[slot]).start()

    def wait_copy(step, slot):
        pltpu.make_async_copy(x_hbm.at[:, pl.ds(step * vb, vb)],
                              xbuf.at[slot], sem.at[slot]).wait()

    start_copy(0, 0)
    start_copy(1, 1)
    acc[...] = jnp.full(acc.shape, _NEG_INF, jnp.float32)
    blk[...] = jnp.zeros(blk.shape, jnp.int32)
    base = base_ref[...]

    def step_fn(i, _):
        slot = jax.lax.rem(i, 2)
        wait_copy(i, slot)
        t = base + (i * vb).astype(jnp.uint32)
        s = _gumbel_from_counter(t) + xbuf[slot]
        a = acc[...]
        acc[...] = jnp.maximum(a, s)
        blk[...] = jnp.where(s > a, i * vb, blk[...])

        @pl.when(i + 2 < nf)
        def _():
            start_copy(i + 2, slot)

        return 0

    jax.lax.fori_loop(0, nf, step_fn, 0)

    t = base[:, :tail] + jnp.uint32(nf * vb)
    st = _gumbel_from_counter(t) + xt_ref[...]
    at = acc[:, :tail]
    acc[:, :tail] = jnp.maximum(at, st)
    blk[:, :tail] = jnp.where(st > at, nf * vb, blk[:, :tail])

    a = acc[...]
    col = jax.lax.broadcasted_iota(jnp.int32, a.shape, 1)
    gidx = blk[...] + col
    rowmax = jnp.max(a, axis=1, keepdims=True)
    cand = jnp.where(a == rowmax, gidx, _INT_MAX)
    o_ref[...] = jnp.min(cand, axis=1, keepdims=True)


@jax.jit
def kernel(inputs):
    b, nv = inputs.shape
    vb = 512
    nf = nv // vb
    tail = nv - nf * vb
    rows = jnp.arange(b, dtype=jnp.int32) * nv
    cols = jnp.arange(vb, dtype=jnp.int32)
    base = (rows[:, None] + cols[None, :] + 42).astype(jnp.uint32)
    x_tail = jax.lax.slice(inputs, (0, nf * vb), (b, nv))
    out = pl.pallas_call(
        functools.partial(_body, vb=vb, nf=nf, tail=tail),
        in_specs=[pl.BlockSpec(memory_space=pltpu.MemorySpace.HBM),
                  pl.BlockSpec((b, vb), lambda: (0, 0)),
                  pl.BlockSpec((b, tail), lambda: (0, 0))],
        out_specs=pl.BlockSpec((b, 1), lambda: (0, 0)),
        out_shape=jax.ShapeDtypeStruct((b, 1), jnp.int32),
        scratch_shapes=[pltpu.VMEM((2, b, vb), jnp.float32),
                        pltpu.VMEM((b, vb), jnp.float32),
                        pltpu.VMEM((b, vb), jnp.int32),
                        pltpu.SemaphoreType.DMA((2,))],
    )(inputs, base, x_tail)
    return out.reshape(b)


# speedup vs baseline: 1.0060x; 1.0060x over previous
import functools

import jax
import jax.numpy as jnp
import numpy as np
from jax.experimental import pallas as pl
from jax.experimental.pallas import tpu as pltpu

_TINY = np.float32(np.finfo(np.float32).tiny)
_NEG_INF = np.float32(-np.inf)
_INT_MAX = np.int32(np.iinfo(np.int32).max)


def _gumbel_from_counter(t):
    ks1 = jnp.uint32(42)
    ks2 = jnp.uint32(0 ^ 42 ^ 0x1BD11BDA)
    ks0 = jnp.uint32(0)
    inj = ((ks1, ks2 + jnp.uint32(1)), (ks2, ks0 + jnp.uint32(2)),
           (ks0, ks1 + jnp.uint32(3)), (ks1, ks2 + jnp.uint32(4)),
           (ks2, ks0 + jnp.uint32(5)))
    rots = ((13, 15, 26, 6), (17, 29, 16, 24))
    x0 = t
    x1 = ((t << jnp.uint32(13)) | (t >> jnp.uint32(19))) ^ t
    first = True
    for g in range(5):
        for r in rots[g & 1]:
            if first:
                first = False
                continue
            x0 = x0 + x1
            x1 = ((x1 << jnp.uint32(r)) | (x1 >> jnp.uint32(32 - r))) ^ x0
        a, b = inj[g]
        x0 = x0 + a
        x1 = x1 + b
    bits = x0 ^ x1
    fb = (bits >> jnp.uint32(9)) | jnp.uint32(0x3F800000)
    f = jax.lax.bitcast_convert_type(fb, jnp.float32) - jnp.float32(1.0)
    u = jnp.maximum(_TINY, f)
    return -jnp.log(-jnp.log(u))


def _body(x_hbm, base_ref, xt_ref, o_ref, xbuf, acc, blk, sem, *, vb, nf,
          tail):
    def start_copy(step, slot):
        pltpu.make_async_copy(x_hbm.at[:, pl.ds(step * vb, vb)],
                              xbuf.at[slot], sem.at[slot]).start()

    def wait_copy(step, slot):
        pltpu.make_async_copy(x_hbm.at[:, pl.ds(step * vb, vb)],
                              xbuf.at[slot], sem.at[slot]).wait()

    start_copy(0, 0)
    start_copy(1, 1)
    acc[...] = jnp.full(acc.shape, _NEG_INF, jnp.float32)
    blk[...] = jnp.zeros(blk.shape, jnp.int32)
    base = base_ref[...]

    def substep(i, slot):
        wait_copy(i, slot)
        t = base + (i * vb).astype(jnp.uint32)
        s = _gumbel_from_counter(t) + xbuf[slot]
        a = acc[...]
        acc[...] = jnp.maximum(a, s)
        blk[...] = jnp.where(s > a, i * vb, blk[...])

        @pl.when(i + 2 < nf)
        def _():
            start_copy(i + 2, slot)

    def step_fn(k, _):
        substep(2 * k, 0)
        substep(2 * k + 1, 1)
        return 0

    jax.lax.fori_loop(0, nf // 2, step_fn, 0)

    t = base[:, :tail] + jnp.uint32(nf * vb)
    st = _gumbel_from_counter(t) + xt_ref[...]
    at = acc[:, :tail]
    acc[:, :tail] = jnp.maximum(at, st)
    blk[:, :tail] = jnp.where(st > at, nf * vb, blk[:, :tail])

    a = acc[...]
    col = jax.lax.broadcasted_iota(jnp.int32, a.shape, 1)
    gidx = blk[...] + col
    rowmax = jnp.max(a, axis=1, keepdims=True)
    cand = jnp.where(a == rowmax, gidx, _INT_MAX)
    o_ref[...] = jnp.min(cand, axis=1, keepdims=True)


@jax.jit
def kernel(inputs):
    b, nv = inputs.shape
    vb = 2048
    nf = nv // vb
    tail = nv - nf * vb
    rows = jnp.arange(b, dtype=jnp.int32) * nv
    cols = jnp.arange(vb, dtype=jnp.int32)
    base = (rows[:, None] + cols[None, :] + 42).astype(jnp.uint32)
    x_tail = jax.lax.slice(inputs, (0, nf * vb), (b, nv))
    out = pl.pallas_call(
        functools.partial(_body, vb=vb, nf=nf, tail=tail),
        in_specs=[pl.BlockSpec(memory_space=pltpu.MemorySpace.HBM),
                  pl.BlockSpec((b, vb), lambda: (0, 0)),
                  pl.BlockSpec((b, tail), lambda: (0, 0))],
        out_specs=pl.BlockSpec((b, 1), lambda: (0, 0)),
        out_shape=jax.ShapeDtypeStruct((b, 1), jnp.int32),
        scratch_shapes=[pltpu.VMEM((2, b, vb), jnp.float32),
                        pltpu.VMEM((b, vb), jnp.float32),
                        pltpu.VMEM((b, vb), jnp.int32),
                        pltpu.SemaphoreType.DMA((2,))],
    )(inputs, base, x_tail)
    return out.reshape(b)


# speedup vs baseline: 1.0120x; 1.0060x over previous
import functools

import jax
import jax.numpy as jnp
import numpy as np
from jax.experimental import pallas as pl
from jax.experimental.pallas import tpu as pltpu

_TINY = np.float32(np.finfo(np.float32).tiny)
_NEG_INF = np.float32(-np.inf)
_INT_MAX = np.int32(np.iinfo(np.int32).max)


def _gumbel_from_counter(t):
    ks1 = jnp.uint32(42)
    ks2 = jnp.uint32(0 ^ 42 ^ 0x1BD11BDA)
    ks0 = jnp.uint32(0)
    inj = ((ks1, ks2 + jnp.uint32(1)), (ks2, ks0 + jnp.uint32(2)),
           (ks0, ks1 + jnp.uint32(3)), (ks1, ks2 + jnp.uint32(4)),
           (ks2, ks0 + jnp.uint32(5)))
    rots = ((13, 15, 26, 6), (17, 29, 16, 24))
    x0 = t
    x1 = ((t << jnp.uint32(13)) | (t >> jnp.uint32(19))) ^ t
    first = True
    for g in range(5):
        for r in rots[g & 1]:
            if first:
                first = False
                continue
            x0 = x0 + x1
            x1 = ((x1 << jnp.uint32(r)) | (x1 >> jnp.uint32(32 - r))) ^ x0
        a, b = inj[g]
        x0 = x0 + a
        x1 = x1 + b
    bits = x0 ^ x1
    fb = (bits >> jnp.uint32(9)) | jnp.uint32(0x3F800000)
    f = jax.lax.bitcast_convert_type(fb, jnp.float32) - jnp.float32(1.0)
    u = jnp.maximum(_TINY, f)
    return -jnp.log(-jnp.log(u))


def _body(x_hbm, base_ref, xt_ref, o_ref, xbuf, acc, blk, sem, *, vb, nf,
          tail):
    def start_copy(step, slot):
        pltpu.make_async_copy(x_hbm.at[:, pl.ds(step * vb, vb)],
                              xbuf.at[slot], sem.at[slot]).start()

    def wait_copy(step, slot):
        pltpu.make_async_copy(x_hbm.at[:, pl.ds(step * vb, vb)],
                              xbuf.at[slot], sem.at[slot]).wait()

    start_copy(0, 0)
    start_copy(1, 1)
    acc[...] = jnp.full(acc.shape, _NEG_INF, jnp.float32)
    blk[...] = jnp.zeros(blk.shape, jnp.int32)
    base = base_ref[...]

    def step_fn(i, _):
        slot = jax.lax.rem(i, 2)
        wait_copy(i, slot)
        t = base + (i * vb).astype(jnp.uint32)
        s = _gumbel_from_counter(t) + xbuf[slot]
        a = acc[...]
        acc[...] = jnp.maximum(a, s)
        blk[...] = jnp.where(s > a, i * vb, blk[...])

        @pl.when(i + 2 < nf)
        def _():
            start_copy(i + 2, slot)

        return 0

    jax.lax.fori_loop(0, nf, step_fn, 0)

    t = base[:, :tail] + jnp.uint32(nf * vb)
    st = _gumbel_from_counter(t) + xt_ref[...]
    at = acc[:, :tail]
    acc[:, :tail] = jnp.maximum(at, st)
    blk[:, :tail] = jnp.where(st > at, nf * vb, blk[:, :tail])

    a = acc[...]
    col = jax.lax.broadcasted_iota(jnp.int32, a.shape, 1)
    gidx = blk[...] + col
    rowmax = jnp.max(a, axis=1, keepdims=True)
    cand = jnp.where(a == rowmax, gidx, _INT_MAX)
    o_ref[...] = jnp.min(cand, axis=1, keepdims=True)


@jax.jit
def kernel(inputs):
    b, nv = inputs.shape
    vb = 1024
    nf = nv // vb
    tail = nv - nf * vb
    rows = jnp.arange(b, dtype=jnp.int32) * nv
    cols = jnp.arange(vb, dtype=jnp.int32)
    base = (rows[:, None] + cols[None, :] + 42).astype(jnp.uint32)
    x_tail = jax.lax.slice(inputs, (0, nf * vb), (b, nv))
    out = pl.pallas_call(
        functools.partial(_body, vb=vb, nf=nf, tail=tail),
        in_specs=[pl.BlockSpec(memory_space=pltpu.MemorySpace.HBM),
                  pl.BlockSpec((b, vb), lambda: (0, 0)),
                  pl.BlockSpec((b, tail), lambda: (0, 0))],
        out_specs=pl.BlockSpec((b, 1), lambda: (0, 0)),
        out_shape=jax.ShapeDtypeStruct((b, 1), jnp.int32),
        scratch_shapes=[pltpu.VMEM((2, b, vb), jnp.float32),
                        pltpu.VMEM((b, vb), jnp.float32),
                        pltpu.VMEM((b, vb), jnp.int32),
                        pltpu.SemaphoreType.DMA((2,))],
    )(inputs, base, x_tail)
    return out.reshape(b)
